# baseline (speedup 1.0000x reference)
import functools

import jax
import jax.numpy as jnp
from jax.experimental import pallas as pl
from jax.experimental.pallas import tpu as pltpu

N = 4096
D = 256
K = 16
INV_T = 10.0
BLOCK = 512
NEG = -3.0
BIG = 4.0
LOG2E = 1.4426950408889634
S = N // 128


def _norm_body(x_ref, o_ref):
    x = x_ref[...]
    n = jnp.maximum(jnp.sqrt(jnp.sum(x * x, axis=-1, keepdims=True)), 1e-12)
    o_ref[...] = x / n


def _sl(x, g):
    return x[:, g * 128:(g + 1) * 128]


def _body(xb_ref, xf_ref, o_ref):
    xb = xb_ref[...]
    xf = xf_ref[...]
    sim = jax.lax.dot_general(
        xb, xf, (((1,), (1,)), ((), ())), preferred_element_type=jnp.float32
    )

    ws = [_sl(sim, g) for g in range(S)]
    t1 = functools.reduce(jnp.maximum, ws)
    cands = [t1]
    tp = t1
    for _lvl in range(3):
        ws = [jnp.where(wg == tp, NEG, wg) for wg in ws]
        tp = functools.reduce(jnp.maximum, ws)
        cands.append(tp)
    cand = jnp.concatenate(cands, axis=-1)

    m0 = jnp.max(t1, axis=-1, keepdims=True)
    w = cand
    t = m0
    for _ in range(K - 1):
        w = jnp.where(w >= t, NEG, w)
        t = jnp.max(w, axis=-1, keepdims=True)

    e_cand = jnp.where(cand >= t, jnp.exp((cand - m0) * INV_T), 0.0)
    s = jnp.sum(e_cand, axis=-1, keepdims=True)

    c1 = INV_T * LOG2E
    bias = m0 * c1 + jnp.log2(s)
    cacc = jnp.zeros((BLOCK, 128), jnp.float32)
    for g in range(S):
        sg = _sl(sim, g)
        ge = sg >= t
        o_ref[:, g * 128:(g + 1) * 128] = jnp.where(
            ge, jnp.exp2(sg * c1 - bias), 0.0
        )
        cacc = cacc + jnp.where(ge, 1.0, 0.0)
    count = jnp.sum(cacc, axis=-1, keepdims=True)

    @pl.when(jnp.any(count > float(K)))
    def _fix():
        def _cond(state):
            _, count_, it = state
            return jnp.logical_and(jnp.any(count_ > float(K)), it < 24)

        def _repair(state):
            t_, count_, it = state
            bad = count_ > float(K)
            macc = functools.reduce(
                jnp.minimum,
                [jnp.where(_sl(sim, g) >= t_, _sl(sim, g), BIG) for g in range(S)],
            )
            m = jnp.min(macc, axis=-1, keepdims=True)
            nacc = functools.reduce(
                jnp.minimum,
                [jnp.where(_sl(sim, g) > m, _sl(sim, g), BIG) for g in range(S)],
            )
            tn = jnp.min(nacc, axis=-1, keepdims=True)
            t2_ = jnp.where(jnp.logical_and(bad, tn < BIG), tn, t_)
            c2acc = functools.reduce(
                jnp.add,
                [jnp.where(_sl(sim, g) >= t2_, 1.0, 0.0) for g in range(S)],
            )
            c2 = jnp.sum(c2acc, axis=-1, keepdims=True)
            return t2_, c2, it + 1

        t2, _, _ = jax.lax.while_loop(_cond, _repair, (t, count, 0))
        eacc = functools.reduce(
            jnp.add,
            [
                jnp.where(
                    _sl(sim, g) >= t2, jnp.exp((_sl(sim, g) - m0) * INV_T), 0.0
                )
                for g in range(S)
            ],
        )
        s2 = jnp.sum(eacc, axis=-1, keepdims=True)
        bias2 = m0 * c1 + jnp.log2(s2)
        for g in range(S):
            sg = _sl(sim, g)
            o_ref[:, g * 128:(g + 1) * 128] = jnp.where(
                sg >= t2, jnp.exp2(sg * c1 - bias2), 0.0
            )


def kernel(X_c):
    Xn = pl.pallas_call(
        _norm_body,
        grid=(4,),
        in_specs=[pl.BlockSpec((N // 4, D), lambda i: (i, 0))],
        out_specs=pl.BlockSpec((N // 4, D), lambda i: (i, 0)),
        out_shape=jax.ShapeDtypeStruct((N, D), jnp.float32),
    )(X_c)
    return pl.pallas_call(
        _body,
        grid=(N // BLOCK,),
        in_specs=[
            pl.BlockSpec((BLOCK, D), lambda i: (i, 0)),
            pl.BlockSpec((N, D), lambda i: (0, 0)),
        ],
        out_specs=pl.BlockSpec((BLOCK, N), lambda i: (i, 0)),
        out_shape=jax.ShapeDtypeStruct((N, N), jnp.float32),
        compiler_params=pltpu.CompilerParams(
            dimension_semantics=("arbitrary",),
        ),
    )(Xn, Xn)


# speedup vs baseline: 1.0058x; 1.0058x over previous
import functools

import jax
import jax.numpy as jnp
from jax.experimental import pallas as pl
from jax.experimental.pallas import tpu as pltpu

N = 4096
D = 256
K = 16
INV_T = 10.0
BLOCK = 512
NEG = -3.0
BIG = 4.0
LOG2E = 1.4426950408889634
S = N // 128


def _norm_body(x_ref, o_ref):
    x = x_ref[...]
    n = jnp.maximum(jnp.sqrt(jnp.sum(x * x, axis=-1, keepdims=True)), 1e-12)
    o_ref[...] = x / n


def _sl(x, g):
    return x[:, g * 128:(g + 1) * 128]


def _body(xb_ref, xf_ref, o_ref):
    xb = xb_ref[...]
    xf = xf_ref[...]
    sim = jax.lax.dot_general(
        xb, xf, (((1,), (1,)), ((), ())), preferred_element_type=jnp.float32
    )

    ws = [_sl(sim, g) for g in range(S)]
    t1 = functools.reduce(jnp.maximum, ws)
    cands = [t1]
    tp = t1
    for _lvl in range(3):
        ws = [jnp.where(wg == tp, NEG, wg) for wg in ws]
        tp = functools.reduce(jnp.maximum, ws)
        cands.append(tp)
    cand = jnp.concatenate(cands, axis=-1)
    t5 = functools.reduce(
        jnp.maximum, [jnp.where(wg == tp, NEG, wg) for wg in ws]
    )

    m0 = jnp.max(t1, axis=-1, keepdims=True)
    w = cand
    t = m0
    for _ in range(K - 1):
        w = jnp.where(w >= t, NEG, w)
        t = jnp.max(w, axis=-1, keepdims=True)

    e_cand = jnp.where(cand >= t, jnp.exp((cand - m0) * INV_T), 0.0)
    s0 = jnp.sum(e_cand, axis=-1, keepdims=True)
    t_ref_init = t
    s_ref_init = s0

    bad5 = jnp.any(t5 >= t, axis=-1, keepdims=True)

    def _fixed_ts():
        cacc = functools.reduce(
            jnp.add, [jnp.where(_sl(sim, g) >= t, 1.0, 0.0) for g in range(S)]
        )
        count = jnp.sum(cacc, axis=-1, keepdims=True)

        def _cond(state):
            _, count_, it = state
            return jnp.logical_and(jnp.any(count_ > float(K)), it < 24)

        def _repair(state):
            t_, count_, it = state
            bad = count_ > float(K)
            macc = functools.reduce(
                jnp.minimum,
                [jnp.where(_sl(sim, g) >= t_, _sl(sim, g), BIG) for g in range(S)],
            )
            m = jnp.min(macc, axis=-1, keepdims=True)
            nacc = functools.reduce(
                jnp.minimum,
                [jnp.where(_sl(sim, g) > m, _sl(sim, g), BIG) for g in range(S)],
            )
            tn = jnp.min(nacc, axis=-1, keepdims=True)
            t2_ = jnp.where(jnp.logical_and(bad, tn < BIG), tn, t_)
            c2acc = functools.reduce(
                jnp.add,
                [jnp.where(_sl(sim, g) >= t2_, 1.0, 0.0) for g in range(S)],
            )
            c2 = jnp.sum(c2acc, axis=-1, keepdims=True)
            return t2_, c2, it + 1

        t2, _, _ = jax.lax.while_loop(_cond, _repair, (t, count, 0))
        eacc = functools.reduce(
            jnp.add,
            [
                jnp.where(
                    _sl(sim, g) >= t2, jnp.exp((_sl(sim, g) - m0) * INV_T), 0.0
                )
                for g in range(S)
            ],
        )
        s2 = jnp.sum(eacc, axis=-1, keepdims=True)
        return t2, s2

    t, s = jax.lax.cond(
        jnp.any(bad5), _fixed_ts, lambda: (t_ref_init, s_ref_init)
    )

    c1 = INV_T * LOG2E
    bias = m0 * c1 + jnp.log2(s)
    for g in range(S):
        sg = _sl(sim, g)
        o_ref[:, g * 128:(g + 1) * 128] = jnp.where(
            sg >= t, jnp.exp2(sg * c1 - bias), 0.0
        )


def kernel(X_c):
    Xn = pl.pallas_call(
        _norm_body,
        grid=(4,),
        in_specs=[pl.BlockSpec((N // 4, D), lambda i: (i, 0))],
        out_specs=pl.BlockSpec((N // 4, D), lambda i: (i, 0)),
        out_shape=jax.ShapeDtypeStruct((N, D), jnp.float32),
    )(X_c)
    return pl.pallas_call(
        _body,
        grid=(N // BLOCK,),
        in_specs=[
            pl.BlockSpec((BLOCK, D), lambda i: (i, 0)),
            pl.BlockSpec((N, D), lambda i: (0, 0)),
        ],
        out_specs=pl.BlockSpec((BLOCK, N), lambda i: (i, 0)),
        out_shape=jax.ShapeDtypeStruct((N, N), jnp.float32),
        compiler_params=pltpu.CompilerParams(
            dimension_semantics=("arbitrary",),
        ),
    )(Xn, Xn)


# speedup vs baseline: 1.1483x; 1.1417x over previous
import functools

import jax
import jax.numpy as jnp
from jax.experimental import pallas as pl
from jax.experimental.pallas import tpu as pltpu

N = 4096
D = 256
K = 16
INV_T = 10.0
BLOCK = 512
NEG = -3.0
BIG = 4.0
LOG2E = 1.4426950408889634
S = N // 128


def _norm_body(x_ref, o_ref):
    x = x_ref[...]
    n = jnp.maximum(jnp.sqrt(jnp.sum(x * x, axis=-1, keepdims=True)), 1e-12)
    o_ref[...] = x / n


def _body(xb_ref, xf_ref, o_ref):
    xb = xb_ref[...]

    neg = jnp.full((BLOCK, 128), NEG, jnp.float32)
    a1, a2, a3, a4, a5 = neg, neg, neg, neg, neg
    sims = []
    for g in range(S):
        v = jax.lax.dot_general(
            xb,
            xf_ref[g * 128:(g + 1) * 128, :],
            (((1,), (1,)), ((), ())),
            preferred_element_type=jnp.float32,
        )
        sims.append(v)
        lo = jnp.minimum(a1, v)
        a1 = jnp.maximum(a1, v)
        lo2 = jnp.minimum(a2, lo)
        a2 = jnp.maximum(a2, lo)
        lo3 = jnp.minimum(a3, lo2)
        a3 = jnp.maximum(a3, lo2)
        lo4 = jnp.minimum(a4, lo3)
        a4 = jnp.maximum(a4, lo3)
        a5 = jnp.maximum(a5, lo4)

    cand = jnp.concatenate([a1, a2, a3, a4], axis=-1)

    m0 = jnp.max(a1, axis=-1, keepdims=True)
    w = cand
    t = m0
    for _ in range(K - 1):
        w = jnp.where(w >= t, NEG, w)
        t = jnp.max(w, axis=-1, keepdims=True)

    e_cand = jnp.where(cand >= t, jnp.exp((cand - m0) * INV_T), 0.0)
    s0 = jnp.sum(e_cand, axis=-1, keepdims=True)
    t_init = t
    s_init = s0

    def _fixed_ts():
        cacc = functools.reduce(
            jnp.add, [jnp.where(sg >= t, 1.0, 0.0) for sg in sims]
        )
        count = jnp.sum(cacc, axis=-1, keepdims=True)

        def _cond(state):
            _, count_, it = state
            return jnp.logical_and(jnp.any(count_ > float(K)), it < 24)

        def _repair(state):
            t_, count_, it = state
            bad = count_ > float(K)
            macc = functools.reduce(
                jnp.minimum,
                [jnp.where(sg >= t_, sg, BIG) for sg in sims],
            )
            m = jnp.min(macc, axis=-1, keepdims=True)
            nacc = functools.reduce(
                jnp.minimum,
                [jnp.where(sg > m, sg, BIG) for sg in sims],
            )
            tn = jnp.min(nacc, axis=-1, keepdims=True)
            t2_ = jnp.where(jnp.logical_and(bad, tn < BIG), tn, t_)
            c2acc = functools.reduce(
                jnp.add, [jnp.where(sg >= t2_, 1.0, 0.0) for sg in sims]
            )
            c2 = jnp.sum(c2acc, axis=-1, keepdims=True)
            return t2_, c2, it + 1

        t2, _, _ = jax.lax.while_loop(_cond, _repair, (t, count, 0))
        eacc = functools.reduce(
            jnp.add,
            [
                jnp.where(sg >= t2, jnp.exp((sg - m0) * INV_T), 0.0)
                for sg in sims
            ],
        )
        s2 = jnp.sum(eacc, axis=-1, keepdims=True)
        return t2, s2

    t, s = jax.lax.cond(
        jnp.any(jnp.any(a5 >= t, axis=-1, keepdims=True)),
        _fixed_ts,
        lambda: (t_init, s_init),
    )

    c1 = INV_T * LOG2E
    bias = m0 * c1 + jnp.log2(s)
    for g in range(S):
        sg = sims[g]
        o_ref[:, g * 128:(g + 1) * 128] = jnp.where(
            sg >= t, jnp.exp2(sg * c1 - bias), 0.0
        )


def kernel(X_c):
    Xn = pl.pallas_call(
        _norm_body,
        grid=(4,),
        in_specs=[pl.BlockSpec((N // 4, D), lambda i: (i, 0))],
        out_specs=pl.BlockSpec((N // 4, D), lambda i: (i, 0)),
        out_shape=jax.ShapeDtypeStruct((N, D), jnp.float32),
    )(X_c)
    return pl.pallas_call(
        _body,
        grid=(N // BLOCK,),
        in_specs=[
            pl.BlockSpec((BLOCK, D), lambda i: (i, 0)),
            pl.BlockSpec((N, D), lambda i: (0, 0)),
        ],
        out_specs=pl.BlockSpec((BLOCK, N), lambda i: (i, 0)),
        out_shape=jax.ShapeDtypeStruct((N, N), jnp.float32),
        compiler_params=pltpu.CompilerParams(
            dimension_semantics=("arbitrary",),
        ),
    )(Xn, Xn)
